# baseline (speedup 1.0000x reference)
import jax
import jax.numpy as jnp
from jax.experimental import pallas as pl


def _pos_body(row_ref, col_ref, out_ref):
    h, w = out_ref.shape[2], out_ref.shape[3]
    d = col_ref.shape[1]
    col = col_ref[0:w, :]
    row = row_ref[0:h, :]
    col_t = col.T
    row_t = row.T
    x_part = jnp.broadcast_to(col_t[:, None, :], (d, h, w))
    y_part = jnp.broadcast_to(row_t[:, :, None], (d, h, w))
    out_ref[0] = jnp.concatenate([x_part, y_part], axis=0)


def kernel(x, row_embed, col_embed):
    b = x.shape[0]
    h, w = x.shape[-2], x.shape[-1]
    d = col_embed.shape[-1]
    out_shape = jax.ShapeDtypeStruct((b, 2 * d, h, w), jnp.float32)
    return pl.pallas_call(
        _pos_body,
        grid=(b,),
        in_specs=[
            pl.BlockSpec(row_embed.shape, lambda i: (0, 0)),
            pl.BlockSpec(col_embed.shape, lambda i: (0, 0)),
        ],
        out_specs=pl.BlockSpec((1, 2 * d, h, w), lambda i: (i, 0, 0, 0)),
        out_shape=out_shape,
    )(row_embed, col_embed)


# speedup vs baseline: 2.5927x; 2.5927x over previous
import jax
import jax.numpy as jnp
from jax.experimental import pallas as pl


def _make_pos_body(h, w):
    def _pos_body(row_ref, col_ref, out_ref):
        d = col_ref.shape[1]
        hw = h * w
        col = col_ref[0:w, :]
        row = row_ref[0:h, :]
        lane = jax.lax.broadcasted_iota(jnp.int32, (w, hw), 1)
        sub_w = jax.lax.broadcasted_iota(jnp.int32, (w, hw), 0)
        sel_col = (lane % w == sub_w).astype(jnp.float32)
        lane_h = jax.lax.broadcasted_iota(jnp.int32, (h, hw), 1)
        sub_h = jax.lax.broadcasted_iota(jnp.int32, (h, hw), 0)
        sel_row = (lane_h // w == sub_h).astype(jnp.float32)
        dn = (((0,), (0,)), ((), ()))
        xp = jax.lax.dot_general(col, sel_col, dn,
                                 preferred_element_type=jnp.float32)
        yp = jax.lax.dot_general(row, sel_row, dn,
                                 preferred_element_type=jnp.float32)
        out_ref[0] = jnp.concatenate([xp, yp], axis=0)
    return _pos_body


def kernel(x, row_embed, col_embed):
    b = x.shape[0]
    h, w = x.shape[-2], x.shape[-1]
    d = col_embed.shape[-1]
    out_flat = pl.pallas_call(
        _make_pos_body(h, w),
        grid=(b,),
        in_specs=[
            pl.BlockSpec(row_embed.shape, lambda i: (0, 0)),
            pl.BlockSpec(col_embed.shape, lambda i: (0, 0)),
        ],
        out_specs=pl.BlockSpec((1, 2 * d, h * w), lambda i: (i, 0, 0)),
        out_shape=jax.ShapeDtypeStruct((b, 2 * d, h * w), jnp.float32),
    )(row_embed, col_embed)
    return out_flat.reshape(b, 2 * d, h, w)


# speedup vs baseline: 2.5996x; 1.0027x over previous
import jax
import jax.numpy as jnp
from jax.experimental import pallas as pl
from jax.experimental.pallas import tpu as pltpu


def _make_pos_body(h, w):
    def _pos_body(row_ref, col_ref, out_ref, scratch_ref):
        d = col_ref.shape[1]

        @pl.when(pl.program_id(0) == 0)
        def _compute():
            col_t = col_ref[0:w, :].T
            row_t = row_ref[0:h, :].T
            xp = jnp.broadcast_to(col_t[:, None, :], (d, h, w))
            yp = jnp.broadcast_to(row_t[:, :, None], (d, h, w))
            pos = jnp.concatenate([xp, yp], axis=0)
            scratch_ref[...] = pos.reshape(2 * d, h * w)

        out_ref[0] = scratch_ref[...]
    return _pos_body


def kernel(x, row_embed, col_embed):
    b = x.shape[0]
    h, w = x.shape[-2], x.shape[-1]
    d = col_embed.shape[-1]
    out_flat = pl.pallas_call(
        _make_pos_body(h, w),
        grid=(b,),
        in_specs=[
            pl.BlockSpec(row_embed.shape, lambda i: (0, 0)),
            pl.BlockSpec(col_embed.shape, lambda i: (0, 0)),
        ],
        out_specs=pl.BlockSpec((1, 2 * d, h * w), lambda i: (i, 0, 0)),
        out_shape=jax.ShapeDtypeStruct((b, 2 * d, h * w), jnp.float32),
        scratch_shapes=[pltpu.VMEM((2 * d, h * w), jnp.float32)],
    )(row_embed, col_embed)
    return out_flat.reshape(b, 2 * d, h, w)


# speedup vs baseline: 2.6854x; 1.0330x over previous
import jax
import jax.numpy as jnp
from jax.experimental import pallas as pl
from jax.experimental.pallas import tpu as pltpu


def _make_body(b, d, h, w):
    def body(row_ref, col_ref, out_ref, scratch, sems):
        col_t = col_ref[0:w, :].T
        row_t = row_ref[0:h, :].T
        xp = jnp.broadcast_to(col_t[:, None, :], (d, h, w)).reshape(d, h * w)
        yp = jnp.broadcast_to(row_t[:, :, None], (d, h, w)).reshape(d, h * w)
        scratch[0:d] = xp
        scratch[d:2 * d] = yp
        copies = [
            pltpu.make_async_copy(scratch, out_ref.at[i], sems.at[i])
            for i in range(b)
        ]
        for c in copies:
            c.start()
        for c in copies:
            c.wait()
    return body


def kernel(x, row_embed, col_embed):
    b = x.shape[0]
    h, w = x.shape[-2], x.shape[-1]
    d = col_embed.shape[-1]
    out_flat = pl.pallas_call(
        _make_body(b, d, h, w),
        in_specs=[
            pl.BlockSpec(memory_space=pltpu.VMEM),
            pl.BlockSpec(memory_space=pltpu.VMEM),
        ],
        out_specs=pl.BlockSpec(memory_space=pltpu.MemorySpace.HBM),
        out_shape=jax.ShapeDtypeStruct((b, 2 * d, h * w), jnp.float32),
        scratch_shapes=[
            pltpu.VMEM((2 * d, h * w), jnp.float32),
            pltpu.SemaphoreType.DMA((b,)),
        ],
    )(row_embed, col_embed)
    return out_flat.reshape(b, 2 * d, h, w)
